# Initial kernel scaffold; baseline (speedup 1.0000x reference)
import jax
import jax.numpy as jnp
from jax.experimental import pallas as pl


def kernel(x, hash_table):
    raise NotImplementedError("write your pallas kernel here")


# speedup vs baseline: 1.1192x; 1.1192x over previous
import functools

import jax
import jax.numpy as jnp
import numpy as np
from jax import lax
from jax.experimental import pallas as pl
from jax.experimental.pallas import tpu as pltpu
from jax.experimental.pallas import tpu_sc as plsc

L = 16
F = 2
T = 2 ** 19
MASK = T - 1
N = 65536
N_min = 16
bb = 1.26

NW = 32
SPW = N // NW
CH = 512
NCH = SPW // CH
NG = CH // 16
NTAP = CH * 16

_PI2J = []
for _j in range(4):
    _v = (_j * 2654435761) % (2 ** 32)
    if _v >= 2 ** 31:
        _v -= 2 ** 32
    _PI2J.append(np.int32(_v))


def _cr_weights(t):
    t2 = t * t
    t3 = t2 * t
    w0 = 0.5 * (2.0 * t2 - t3 - t)
    w1 = 0.5 * (3.0 * t3 - 5.0 * t2 + 2.0)
    w2 = 0.5 * (t + 4.0 * t2 - 3.0 * t3)
    w3 = 0.5 * (t3 - t2)
    return (w0, w1, w2, w3)


_mesh = plsc.VectorSubcoreMesh(core_axis_name="c", subcore_axis_name="s")


@functools.partial(
    pl.kernel,
    mesh=_mesh,
    compiler_params=pltpu.CompilerParams(needs_layout_passes=False),
    out_type=jax.ShapeDtypeStruct((N * L * F,), jnp.float32),
    scratch_types=[
        pltpu.VMEM((CH,), jnp.float32),
        pltpu.VMEM((CH,), jnp.float32),
        pltpu.VMEM((NTAP,), jnp.int32),
        pltpu.VMEM((NTAP,), jnp.int32),
        pltpu.VMEM((NTAP,), jnp.float32),
        pltpu.VMEM((NTAP,), jnp.float32),
        pltpu.VMEM((CH * L * F,), jnp.float32),
        pltpu.SemaphoreType.DMA,
    ],
)
def _sc_encode(xn_hbm, tab_hbm, out_hbm, x0_v, x1_v, ix0_v, ix1_v,
               st0_v, st1_v, out_v, sem):
    wid = lax.axis_index("s") * 2 + lax.axis_index("c")
    iota = lax.iota(jnp.int32, 16)
    iota32 = iota * 32

    def chunk_body(ch, carry0):
        base = wid * SPW + ch * CH

        def level_body(l, carry1):
            pltpu.sync_copy(xn_hbm.at[l * 2, pl.ds(base, CH)], x0_v)
            pltpu.sync_copy(xn_hbm.at[l * 2 + 1, pl.ds(base, CH)], x1_v)
            lT2 = l * (2 * T)

            def idx_body(g, carry2):
                xn0 = x0_v[pl.ds(g * 16, 16)]
                xn1 = x1_v[pl.ds(g * 16, 16)]
                X = xn0.astype(jnp.int32)
                YP = xn1.astype(jnp.int32) * _PI2J[1]
                for t in range(16):
                    i, j = t // 4, t % 4
                    h = ((X + i) ^ (YP + _PI2J[j])) & MASK
                    e0 = h * 2 + lT2
                    ix0_v[pl.ds(g * 256 + t * 16, 16)] = e0
                    ix1_v[pl.ds(g * 256 + t * 16, 16)] = e0 + 1
                return carry2

            lax.fori_loop(jnp.int32(0), jnp.int32(NG), idx_body, jnp.int32(0))

            handles = []
            for r in range(NTAP // 128):
                handles.append(pltpu.async_copy(
                    tab_hbm.at[ix0_v.at[pl.ds(r * 128, 128)]],
                    st0_v.at[pl.ds(r * 128, 128)], sem))
                handles.append(pltpu.async_copy(
                    tab_hbm.at[ix1_v.at[pl.ds(r * 128, 128)]],
                    st1_v.at[pl.ds(r * 128, 128)], sem))
            for h_ in handles:
                h_.wait()

            def interp_body(g, carry2):
                xn0 = x0_v[pl.ds(g * 16, 16)]
                xn1 = x1_v[pl.ds(g * 16, 16)]
                X = xn0.astype(jnp.int32)
                Y = xn1.astype(jnp.int32)
                tx = xn0 - X.astype(jnp.float32)
                ty = xn1 - Y.astype(jnp.float32)
                wx = _cr_weights(tx)
                wy = _cr_weights(ty)
                acc0 = jnp.zeros((16,), jnp.float32)
                acc1 = jnp.zeros((16,), jnp.float32)
                for t in range(16):
                    i, j = t // 4, t % 4
                    v0 = st0_v[pl.ds(g * 256 + t * 16, 16)]
                    v1 = st1_v[pl.ds(g * 256 + t * 16, 16)]
                    P = wx[i] * wy[j]
                    acc0 = acc0 + P * v0
                    acc1 = acc1 + P * v1
                off = g * (16 * L * F) + l * 2
                plsc.store_scatter(out_v, [iota32 + off], acc0)
                plsc.store_scatter(out_v, [iota32 + (off + 1)], acc1)
                return carry2

            lax.fori_loop(jnp.int32(0), jnp.int32(NG), interp_body, jnp.int32(0))
            return carry1

        lax.fori_loop(jnp.int32(0), jnp.int32(L), level_body, jnp.int32(0))
        pltpu.sync_copy(out_v, out_hbm.at[pl.ds(base * (L * F), CH * L * F)])
        return carry0

    lax.fori_loop(jnp.int32(0), jnp.int32(NCH), chunk_body, jnp.int32(0))


def kernel(x, hash_table):
    x = x.astype(jnp.float32)
    hash_table = hash_table.astype(jnp.float32)
    nl = jnp.floor(N_min * (bb ** jnp.arange(L, dtype=jnp.float32)))
    xn = jnp.transpose(x[:, :, None] * nl[None, None, :], (2, 1, 0)).reshape(L * 2, N)
    tab = jnp.transpose(hash_table, (2, 0, 1)).reshape(L * T * F)
    return _sc_encode(xn, tab).reshape(N, L * F)


# speedup vs baseline: 4.0577x; 3.6255x over previous
import functools

import jax
import jax.numpy as jnp
import numpy as np
from jax import lax
from jax.experimental import pallas as pl
from jax.experimental.pallas import tpu as pltpu
from jax.experimental.pallas import tpu_sc as plsc

L = 16
F = 2
T = 2 ** 19
MASK = T - 1
N = 65536
N_min = 16
bb = 1.26

NW = 32
SPW = N // NW
CH = 512
NCH = SPW // CH
NG = CH // 16
NTAP = CH * 16

_PI2J = []
for _j in range(4):
    _v = (_j * 2654435761) % (2 ** 32)
    if _v >= 2 ** 31:
        _v -= 2 ** 32
    _PI2J.append(np.int32(_v))


def _cr_weights(t):
    t2 = t * t
    t3 = t2 * t
    w0 = 0.5 * (2.0 * t2 - t3 - t)
    w1 = 0.5 * (3.0 * t3 - 5.0 * t2 + 2.0)
    w2 = 0.5 * (t + 4.0 * t2 - 3.0 * t3)
    w3 = 0.5 * (t3 - t2)
    return (w0, w1, w2, w3)


_mesh = plsc.VectorSubcoreMesh(core_axis_name="c", subcore_axis_name="s")


@functools.partial(
    pl.kernel,
    mesh=_mesh,
    compiler_params=pltpu.CompilerParams(needs_layout_passes=False),
    out_type=jax.ShapeDtypeStruct((N * L * F,), jnp.float32),
    scratch_types=[
        pltpu.VMEM((CH,), jnp.float32),
        pltpu.VMEM((CH,), jnp.float32),
        pltpu.VMEM((NTAP,), jnp.int32),
        pltpu.VMEM((NTAP,), jnp.int32),
        pltpu.VMEM((NTAP,), jnp.float32),
        pltpu.VMEM((NTAP,), jnp.float32),
        pltpu.VMEM((CH * L * F,), jnp.float32),
        pltpu.SemaphoreType.DMA,
    ],
)
def _sc_encode(xn_hbm, tab_hbm, out_hbm, x0_v, x1_v, ix0_v, ix1_v,
               st0_v, st1_v, out_v, sem):
    wid = lax.axis_index("s") * 2 + lax.axis_index("c")
    iota = lax.iota(jnp.int32, 16)
    iota32 = iota * 32

    def chunk_body(ch, carry0):
        base = wid * SPW + ch * CH

        def level_body(l, carry1):
            pltpu.sync_copy(xn_hbm.at[l * 2, pl.ds(base, CH)], x0_v)
            pltpu.sync_copy(xn_hbm.at[l * 2 + 1, pl.ds(base, CH)], x1_v)

            def idx_body(g, carry2):
                xn0 = x0_v[pl.ds(g * 16, 16)]
                xn1 = x1_v[pl.ds(g * 16, 16)]
                X = xn0.astype(jnp.int32)
                YP = xn1.astype(jnp.int32) * _PI2J[1]
                for t in range(16):
                    i, j = t // 4, t % 4
                    h = ((X + i) ^ (YP + _PI2J[j])) & MASK
                    e0 = h * (F * L) + l
                    ix0_v[pl.ds(g * 256 + t * 16, 16)] = e0
                    ix1_v[pl.ds(g * 256 + t * 16, 16)] = e0 + L
                return carry2

            lax.fori_loop(jnp.int32(0), jnp.int32(NG), idx_body, jnp.int32(0))

            handles = []
            for r in range(NTAP // 128):
                handles.append(pltpu.async_copy(
                    tab_hbm.at[ix0_v.at[pl.ds(r * 128, 128)]],
                    st0_v.at[pl.ds(r * 128, 128)], sem))
                handles.append(pltpu.async_copy(
                    tab_hbm.at[ix1_v.at[pl.ds(r * 128, 128)]],
                    st1_v.at[pl.ds(r * 128, 128)], sem))
            for h_ in handles:
                h_.wait()

            def interp_body(g, carry2):
                xn0 = x0_v[pl.ds(g * 16, 16)]
                xn1 = x1_v[pl.ds(g * 16, 16)]
                X = xn0.astype(jnp.int32)
                Y = xn1.astype(jnp.int32)
                tx = xn0 - X.astype(jnp.float32)
                ty = xn1 - Y.astype(jnp.float32)
                wx = _cr_weights(tx)
                wy = _cr_weights(ty)
                acc0 = jnp.zeros((16,), jnp.float32)
                acc1 = jnp.zeros((16,), jnp.float32)
                for t in range(16):
                    i, j = t // 4, t % 4
                    v0 = st0_v[pl.ds(g * 256 + t * 16, 16)]
                    v1 = st1_v[pl.ds(g * 256 + t * 16, 16)]
                    P = wx[i] * wy[j]
                    acc0 = acc0 + P * v0
                    acc1 = acc1 + P * v1
                off = g * (16 * L * F) + l * 2
                plsc.store_scatter(out_v, [iota32 + off], acc0)
                plsc.store_scatter(out_v, [iota32 + (off + 1)], acc1)
                return carry2

            lax.fori_loop(jnp.int32(0), jnp.int32(NG), interp_body, jnp.int32(0))
            return carry1

        lax.fori_loop(jnp.int32(0), jnp.int32(L), level_body, jnp.int32(0))
        pltpu.sync_copy(out_v, out_hbm.at[pl.ds(base * (L * F), CH * L * F)])
        return carry0

    lax.fori_loop(jnp.int32(0), jnp.int32(NCH), chunk_body, jnp.int32(0))


def kernel(x, hash_table):
    x = x.astype(jnp.float32)
    hash_table = hash_table.astype(jnp.float32)
    nl = jnp.floor(N_min * (bb ** jnp.arange(L, dtype=jnp.float32)))
    xn = jnp.transpose(x[:, :, None] * nl[None, None, :], (2, 1, 0)).reshape(L * 2, N)
    tab = hash_table.reshape(T * F * L)
    return _sc_encode(xn, tab).reshape(N, L * F)


# speedup vs baseline: 4.1078x; 1.0123x over previous
import functools

import jax
import jax.numpy as jnp
import numpy as np
from jax import lax
from jax.experimental import pallas as pl
from jax.experimental.pallas import tpu as pltpu
from jax.experimental.pallas import tpu_sc as plsc

L = 16
F = 2
T = 2 ** 19
MASK = T - 1
N = 65536
N_min = 16
bb = 1.26

NW = 32
SPW = N // NW
CH = 512
NCH = SPW // CH
NG = CH // 16
NTAP = CH * 16

_PI2J = []
for _j in range(4):
    _v = (_j * 2654435761) % (2 ** 32)
    if _v >= 2 ** 31:
        _v -= 2 ** 32
    _PI2J.append(np.int32(_v))


def _cr_weights(t):
    t2 = t * t
    t3 = t2 * t
    w0 = 0.5 * (2.0 * t2 - t3 - t)
    w1 = 0.5 * (3.0 * t3 - 5.0 * t2 + 2.0)
    w2 = 0.5 * (t + 4.0 * t2 - 3.0 * t3)
    w3 = 0.5 * (t3 - t2)
    return (w0, w1, w2, w3)


_mesh = plsc.VectorSubcoreMesh(core_axis_name="c", subcore_axis_name="s")


@functools.partial(
    pl.kernel,
    mesh=_mesh,
    compiler_params=pltpu.CompilerParams(needs_layout_passes=False),
    out_type=jax.ShapeDtypeStruct((N * L * F,), jnp.float32),
    scratch_types=[
        pltpu.VMEM((L,), jnp.float32),
        pltpu.VMEM((CH,), jnp.float32),
        pltpu.VMEM((CH,), jnp.float32),
        pltpu.VMEM((NTAP,), jnp.int32),
        pltpu.VMEM((NTAP,), jnp.int32),
        pltpu.VMEM((NTAP,), jnp.float32),
        pltpu.VMEM((NTAP,), jnp.float32),
        pltpu.VMEM((CH * L * F,), jnp.float32),
        pltpu.SemaphoreType.DMA,
    ],
)
def _sc_encode(x_hbm, nl_hbm, tab_hbm, out_hbm, nl_s, x0_v, x1_v, ix0_v, ix1_v,
               st0_v, st1_v, out_v, sem):
    wid = lax.axis_index("s") * 2 + lax.axis_index("c")
    iota = lax.iota(jnp.int32, 16)
    iota32 = iota * 32
    pltpu.sync_copy(nl_hbm, nl_s)

    def chunk_body(ch, carry0):
        base = wid * SPW + ch * CH
        pltpu.sync_copy(x_hbm.at[jnp.int32(0), pl.ds(base, CH)], x0_v)
        pltpu.sync_copy(x_hbm.at[jnp.int32(1), pl.ds(base, CH)], x1_v)

        def level_body(l, carry1):
            lvec = jnp.broadcast_to(l, (16,)).astype(jnp.int32)
            nl = plsc.load_gather(nl_s, [lvec])

            def idx_body(g, carry2):
                xn0 = x0_v[pl.ds(g * 16, 16)] * nl
                xn1 = x1_v[pl.ds(g * 16, 16)] * nl
                X = xn0.astype(jnp.int32)
                YP = xn1.astype(jnp.int32) * _PI2J[1]
                for t in range(16):
                    i, j = t // 4, t % 4
                    h = ((X + i) ^ (YP + _PI2J[j])) & MASK
                    e0 = h * (F * L) + l
                    ix0_v[pl.ds(g * 256 + t * 16, 16)] = e0
                    ix1_v[pl.ds(g * 256 + t * 16, 16)] = e0 + L
                return carry2

            lax.fori_loop(jnp.int32(0), jnp.int32(NG), idx_body, jnp.int32(0))

            handles = []
            for r in range(NTAP // 128):
                handles.append(pltpu.async_copy(
                    tab_hbm.at[ix0_v.at[pl.ds(r * 128, 128)]],
                    st0_v.at[pl.ds(r * 128, 128)], sem))
                handles.append(pltpu.async_copy(
                    tab_hbm.at[ix1_v.at[pl.ds(r * 128, 128)]],
                    st1_v.at[pl.ds(r * 128, 128)], sem))
            for h_ in handles:
                h_.wait()

            def interp_body(g, carry2):
                xn0 = x0_v[pl.ds(g * 16, 16)] * nl
                xn1 = x1_v[pl.ds(g * 16, 16)] * nl
                X = xn0.astype(jnp.int32)
                Y = xn1.astype(jnp.int32)
                tx = xn0 - X.astype(jnp.float32)
                ty = xn1 - Y.astype(jnp.float32)
                wx = _cr_weights(tx)
                wy = _cr_weights(ty)
                acc0 = jnp.zeros((16,), jnp.float32)
                acc1 = jnp.zeros((16,), jnp.float32)
                for t in range(16):
                    i, j = t // 4, t % 4
                    v0 = st0_v[pl.ds(g * 256 + t * 16, 16)]
                    v1 = st1_v[pl.ds(g * 256 + t * 16, 16)]
                    P = wx[i] * wy[j]
                    acc0 = acc0 + P * v0
                    acc1 = acc1 + P * v1
                off = g * (16 * L * F) + l * 2
                plsc.store_scatter(out_v, [iota32 + off], acc0)
                plsc.store_scatter(out_v, [iota32 + (off + 1)], acc1)
                return carry2

            lax.fori_loop(jnp.int32(0), jnp.int32(NG), interp_body, jnp.int32(0))
            return carry1

        lax.fori_loop(jnp.int32(0), jnp.int32(L), level_body, jnp.int32(0))
        pltpu.sync_copy(out_v, out_hbm.at[pl.ds(base * (L * F), CH * L * F)])
        return carry0

    lax.fori_loop(jnp.int32(0), jnp.int32(NCH), chunk_body, jnp.int32(0))


def kernel(x, hash_table):
    x = x.astype(jnp.float32)
    hash_table = hash_table.astype(jnp.float32)
    nl = jnp.floor(N_min * (bb ** jnp.arange(L, dtype=jnp.float32)))
    tab = hash_table.reshape(T * F * L)
    return _sc_encode(x.T, nl, tab).reshape(N, L * F)


# speedup vs baseline: 4.1803x; 1.0177x over previous
import functools

import jax
import jax.numpy as jnp
import numpy as np
from jax import lax
from jax.experimental import pallas as pl
from jax.experimental.pallas import tpu as pltpu
from jax.experimental.pallas import tpu_sc as plsc

L = 16
F = 2
T = 2 ** 19
MASK = T - 1
N = 65536
N_min = 16
bb = 1.26

NW = 32
SPW = N // NW
CH = 512
NCH = SPW // CH
NG = CH // 16
NTAP = CH * 16
NSEG = NTAP // 128

_PI2J = []
for _j in range(4):
    _v = (_j * 2654435761) % (2 ** 32)
    if _v >= 2 ** 31:
        _v -= 2 ** 32
    _PI2J.append(np.int32(_v))


def _cr_weights(t):
    t2 = t * t
    t3 = t2 * t
    w0 = 0.5 * (2.0 * t2 - t3 - t)
    w1 = 0.5 * (3.0 * t3 - 5.0 * t2 + 2.0)
    w2 = 0.5 * (t + 4.0 * t2 - 3.0 * t3)
    w3 = 0.5 * (t3 - t2)
    return (w0, w1, w2, w3)


_mesh = plsc.VectorSubcoreMesh(core_axis_name="c", subcore_axis_name="s")


@functools.partial(
    pl.kernel,
    mesh=_mesh,
    compiler_params=pltpu.CompilerParams(needs_layout_passes=False),
    out_type=jax.ShapeDtypeStruct((N * L * F,), jnp.float32),
    scratch_types=[
        pltpu.VMEM((L,), jnp.float32),
        pltpu.VMEM((CH,), jnp.float32),
        pltpu.VMEM((CH,), jnp.float32),
        pltpu.VMEM((NTAP,), jnp.int32),
        pltpu.VMEM((NTAP,), jnp.int32),
        pltpu.VMEM((NTAP,), jnp.float32),
        pltpu.VMEM((NTAP,), jnp.float32),
        pltpu.VMEM((NTAP,), jnp.int32),
        pltpu.VMEM((NTAP,), jnp.int32),
        pltpu.VMEM((NTAP,), jnp.float32),
        pltpu.VMEM((NTAP,), jnp.float32),
        pltpu.VMEM((CH * L * F,), jnp.float32),
        pltpu.SemaphoreType.DMA,
        pltpu.SemaphoreType.DMA,
    ],
)
def _sc_encode(x_hbm, nl_hbm, tab_hbm, out_hbm, nl_s, x0_v, x1_v,
               ax0_v, ax1_v, as0_v, as1_v, bx0_v, bx1_v, bs0_v, bs1_v,
               out_v, sem_a, sem_b):
    wid = lax.axis_index("s") * 2 + lax.axis_index("c")
    iota = lax.iota(jnp.int32, 16)
    iota32 = iota * 32
    pltpu.sync_copy(nl_hbm, nl_s)

    def nl_of(l):
        lvec = jnp.broadcast_to(l, (16,)).astype(jnp.int32)
        return plsc.load_gather(nl_s, [lvec])

    def phase_a(l, ix0_v, ix1_v):
        nl = nl_of(l)

        def idx_body(g, carry):
            xn0 = x0_v[pl.ds(g * 16, 16)] * nl
            xn1 = x1_v[pl.ds(g * 16, 16)] * nl
            X = xn0.astype(jnp.int32)
            YP = xn1.astype(jnp.int32) * _PI2J[1]
            for t in range(16):
                i, j = t // 4, t % 4
                h = ((X + i) ^ (YP + _PI2J[j])) & MASK
                e0 = h * (F * L) + l
                ix0_v[pl.ds(g * 256 + t * 16, 16)] = e0
                ix1_v[pl.ds(g * 256 + t * 16, 16)] = e0 + L
            return carry

        lax.fori_loop(jnp.int32(0), jnp.int32(NG), idx_body, jnp.int32(0))

    def fire(ix0_v, ix1_v, st0_v, st1_v, sem):
        pltpu.async_copy(tab_hbm.at[ix0_v], st0_v, sem)
        pltpu.async_copy(tab_hbm.at[ix1_v], st1_v, sem)

    def drain(st0_v, st1_v, sem):
        pltpu.make_async_copy(tab_hbm.at[pl.ds(0, NTAP)], st0_v, sem).wait()
        pltpu.make_async_copy(tab_hbm.at[pl.ds(0, NTAP)], st1_v, sem).wait()

    def phase_b(l, st0_v, st1_v):
        nl = nl_of(l)

        def interp_body(g, carry):
            xn0 = x0_v[pl.ds(g * 16, 16)] * nl
            xn1 = x1_v[pl.ds(g * 16, 16)] * nl
            X = xn0.astype(jnp.int32)
            Y = xn1.astype(jnp.int32)
            tx = xn0 - X.astype(jnp.float32)
            ty = xn1 - Y.astype(jnp.float32)
            wx = _cr_weights(tx)
            wy = _cr_weights(ty)
            acc0 = jnp.zeros((16,), jnp.float32)
            acc1 = jnp.zeros((16,), jnp.float32)
            for t in range(16):
                i, j = t // 4, t % 4
                v0 = st0_v[pl.ds(g * 256 + t * 16, 16)]
                v1 = st1_v[pl.ds(g * 256 + t * 16, 16)]
                P = wx[i] * wy[j]
                acc0 = acc0 + P * v0
                acc1 = acc1 + P * v1
            off = g * (16 * L * F) + l * 2
            plsc.store_scatter(out_v, [iota32 + off], acc0)
            plsc.store_scatter(out_v, [iota32 + (off + 1)], acc1)
            return carry

        lax.fori_loop(jnp.int32(0), jnp.int32(NG), interp_body, jnp.int32(0))

    def chunk_body(ch, carry0):
        base = wid * SPW + ch * CH
        pltpu.sync_copy(x_hbm.at[jnp.int32(0), pl.ds(base, CH)], x0_v)
        pltpu.sync_copy(x_hbm.at[jnp.int32(1), pl.ds(base, CH)], x1_v)

        def pair_body(k, carry1):
            l0 = k * 2
            l1 = l0 + 1
            phase_a(l0, ax0_v, ax1_v)
            fire(ax0_v, ax1_v, as0_v, as1_v, sem_a)
            phase_a(l1, bx0_v, bx1_v)
            fire(bx0_v, bx1_v, bs0_v, bs1_v, sem_b)
            drain(as0_v, as1_v, sem_a)
            phase_b(l0, as0_v, as1_v)
            drain(bs0_v, bs1_v, sem_b)
            phase_b(l1, bs0_v, bs1_v)
            return carry1

        lax.fori_loop(jnp.int32(0), jnp.int32(L // 2), pair_body, jnp.int32(0))

        pltpu.sync_copy(out_v, out_hbm.at[pl.ds(base * (L * F), CH * L * F)])
        return carry0

    lax.fori_loop(jnp.int32(0), jnp.int32(NCH), chunk_body, jnp.int32(0))


def kernel(x, hash_table):
    x = x.astype(jnp.float32)
    hash_table = hash_table.astype(jnp.float32)
    nl = jnp.floor(N_min * (bb ** jnp.arange(L, dtype=jnp.float32)))
    tab = hash_table.reshape(T * F * L)
    return _sc_encode(x.T, nl, tab).reshape(N, L * F)
